# baseline (speedup 1.0000x reference)
import functools
import math

import jax
import jax.numpy as jnp
from jax import lax
from jax.experimental import pallas as pl
from jax.experimental.pallas import tpu as pltpu
from jax.experimental.pallas import tpu_sc as plsc

D_MODEL = 64
SCALE = math.sqrt(D_MODEL)

_info = plsc.get_sparse_core_info()
NC = _info.num_cores
NS = _info.num_subcores
NW = NC * NS
L = _info.num_lanes

ROWS = 4096
SEQ = 200
BB = ROWS // NW
DB = D_MODEL // 8
SB = SEQ // 8
NG = 4
NT = 2
NSUP = SEQ // NG


def _body(x4_ref, table_ref, out_ref, idx_v, sidx, par, gbuf, tbuf, gsem, osem):
    wid = lax.axis_index("s") * NC + lax.axis_index("c")
    pltpu.sync_copy(x4_ref.at[:, wid], idx_v)

    def prep(s, g):
        for v in range(BB // L):
            sl = pl.ds(v * L, L)
            raw = idx_v[s // 8, s % 8, sl]
            sidx[g, sl] = raw >> 1
            par[g, sl] = (raw & 1) << 6

    def gather(s, g):
        return pltpu.make_async_copy(
            table_ref.at[sidx.at[g]], gbuf.at[g], gsem.at[g]
        )

    def store(s, t):
        return pltpu.make_async_copy(
            tbuf.at[t], out_ref.at[s, :, wid], osem.at[t]
        )

    for g in range(NG):
        prep(g, g)
        gather(g, g).start()

    iota = lax.iota(jnp.int32, L)

    def super_it(sup, carry):
        for k in range(NG):
            s = sup * NG + k
            t = k % NT
            gather(s, k).wait()

            def wait_store():
                store(s, t).wait()

            if k >= NT:
                wait_store()
            else:
                pl.when(sup > 0)(wait_store)

            def bgf(bg, c1):
                b0 = bg * L
                rowv = b0 + iota
                parv = par[k, pl.ds(b0, L)]

                @plsc.parallel_loop(0, D_MODEL, unroll=8)
                def _(d):
                    dcol = (d + iota) & (D_MODEL - 1)
                    v = plsc.load_gather(gbuf.at[k], [rowv, dcol + parv])
                    plsc.store_scatter(
                        tbuf.at[t],
                        [dcol >> 3, dcol & 7, rowv],
                        v * SCALE,
                    )
                return c1

            lax.fori_loop(0, BB // L, bgf, 0)

            store(s, t).start()

            @pl.when(sup < NSUP - 1)
            def _():
                prep(s + NG, k)
                gather(s + NG, k).start()

        return carry

    lax.fori_loop(0, NSUP, super_it, 0)

    for t in range(NT):
        store(SEQ - NT + t, t).wait()


@functools.partial(
    pl.kernel,
    mesh=plsc.VectorSubcoreMesh(core_axis_name="c", subcore_axis_name="s"),
    out_type=jax.ShapeDtypeStruct((SEQ, DB, NW, 8, BB), jnp.float32),
    scratch_types=[
        pltpu.VMEM((SB, 8, BB), jnp.int32),
        pltpu.VMEM((NG, BB), jnp.int32),
        pltpu.VMEM((NG, BB), jnp.int32),
        pltpu.VMEM((NG, BB, 2 * D_MODEL), jnp.float32),
        pltpu.VMEM((NT, DB, 8, BB), jnp.float32),
        pltpu.SemaphoreType.DMA((NG,)),
        pltpu.SemaphoreType.DMA((NT,)),
    ],
    compiler_params=pltpu.CompilerParams(
        use_tc_tiling_on_sc=True, needs_layout_passes=False
    ),
)
def _emb_lookup(x4_ref, table_ref, out_ref, idx_v, sidx, par, gbuf, tbuf, gsem, osem):
    _body(x4_ref, table_ref, out_ref, idx_v, sidx, par, gbuf, tbuf, gsem, osem)


def kernel(x, table):
    x4 = jnp.transpose(
        jnp.reshape(jnp.transpose(x.astype(jnp.int32)), (SB, 8, NW, BB)),
        (0, 2, 1, 3),
    )
    t2 = jnp.reshape(table, (500000, 2 * D_MODEL))
    out5 = _emb_lookup(x4, t2)
    return jnp.transpose(out5, (2, 4, 0, 1, 3)).reshape(ROWS, SEQ, D_MODEL)


# speedup vs baseline: 1.1336x; 1.1336x over previous
import functools
import math

import jax
import jax.numpy as jnp
from jax import lax
from jax.experimental import pallas as pl
from jax.experimental.pallas import tpu as pltpu
from jax.experimental.pallas import tpu_sc as plsc

D_MODEL = 64
SCALE = math.sqrt(D_MODEL)

_info = plsc.get_sparse_core_info()
NC = _info.num_cores
NS = _info.num_subcores
NW = NC * NS
L = _info.num_lanes

ROWS = 4096
SEQ = 200
BB = ROWS // NW
DB = D_MODEL // 8
SB = SEQ // 8
NG = 4
NT = 2
NSUP = SEQ // NG


def _body(x4_ref, table_ref, out_ref, idx_v, gbuf, tbuf, gsem, osem):
    wid = lax.axis_index("s") * NC + lax.axis_index("c")
    pltpu.sync_copy(x4_ref.at[:, wid], idx_v)

    def gather(s, g):
        return pltpu.make_async_copy(
            table_ref.at[idx_v.at[s // 8, s % 8]], gbuf.at[g], gsem.at[g]
        )

    def store(s, t):
        return pltpu.make_async_copy(
            tbuf.at[t], out_ref.at[s, :, wid], osem.at[t]
        )

    for g in range(NG):
        gather(g, g).start()

    iota = lax.iota(jnp.int32, L)

    def super_it(sup, carry):
        for k in range(NG):
            s = sup * NG + k
            t = k % NT
            gather(s, k).wait()

            def wait_store():
                store(s, t).wait()

            if k >= NT:
                wait_store()
            else:
                pl.when(sup > 0)(wait_store)

            @plsc.parallel_loop(0, D_MODEL * (BB // L), unroll=8)
            def _(n):
                d = n >> 3
                b0 = (n & 7) * L
                rowv = b0 + iota
                colv = (d + iota) & (D_MODEL - 1)
                v = plsc.load_gather(gbuf.at[k], [rowv, colv])
                plsc.store_scatter(
                    tbuf.at[t],
                    [colv >> 3, colv & 7, rowv],
                    v * SCALE,
                )

            store(s, t).start()

            @pl.when(sup < NSUP - 1)
            def _():
                gather(s + NG, k).start()

        return carry

    lax.fori_loop(0, NSUP, super_it, 0)

    for t in range(NT):
        store(SEQ - NT + t, t).wait()


@functools.partial(
    pl.kernel,
    mesh=plsc.VectorSubcoreMesh(core_axis_name="c", subcore_axis_name="s"),
    out_type=jax.ShapeDtypeStruct((SEQ, DB, NW, 8, BB), jnp.float32),
    scratch_types=[
        pltpu.VMEM((SB, 8, BB), jnp.int32),
        pltpu.VMEM((NG, BB, D_MODEL), jnp.float32),
        pltpu.VMEM((NT, DB, 8, BB), jnp.float32),
        pltpu.SemaphoreType.DMA((NG,)),
        pltpu.SemaphoreType.DMA((NT,)),
    ],
    compiler_params=pltpu.CompilerParams(
        use_tc_tiling_on_sc=False, needs_layout_passes=False
    ),
)
def _emb_lookup(x4_ref, table_ref, out_ref, idx_v, gbuf, tbuf, gsem, osem):
    _body(x4_ref, table_ref, out_ref, idx_v, gbuf, tbuf, gsem, osem)


def kernel(x, table):
    x4 = jnp.transpose(
        jnp.reshape(jnp.transpose(x.astype(jnp.int32)), (SB, 8, NW, BB)),
        (0, 2, 1, 3),
    )
    out5 = _emb_lookup(x4, table)
    return jnp.transpose(out5, (2, 4, 0, 1, 3)).reshape(ROWS, SEQ, D_MODEL)
